# baseline (speedup 1.0000x reference)
import jax
import jax.numpy as jnp
from jax.experimental import pallas as pl
from jax.experimental.pallas import tpu as pltpu
from jax.experimental.pallas import tpu_sc as plsc

D = 100000
B = 128
_I32MAX = jnp.iinfo(jnp.int32).max


CW = 4096
NCH = (D + CW - 1) // CW
LAST = D - (NCH - 1) * CW
NBUF = 4
LA = NBUF - 1


def _u_copy(u_hbm, ub, su, j):
    slot = jax.lax.rem(j, NBUF)
    return pltpu.make_async_copy(
        u_hbm.at[:, pl.ds(j * CW, CW)], ub.at[slot], su.at[slot])


def _u_copy_last(u_hbm, ubl, sul):
    return pltpu.make_async_copy(
        u_hbm.at[:, pl.ds((NCH - 1) * CW, LAST)], ubl, sul)


def _tc_body(pb_ref, pfull_ref, u_hbm, y_ref, lp_ref,
             ub, ubl, su, sul, bk, bi, bp):
    i = pl.program_id(0)
    slot = jax.lax.rem(i, NBUF)

    @pl.when(i == 0)
    def _prologue():
        bk[...] = jnp.full((B, 1), jnp.inf, jnp.float32)
        bi[...] = jnp.zeros((B, 1), jnp.int32)
        bp[...] = jnp.zeros((B, 1), jnp.float32)
        for j in range(LA):
            _u_copy(u_hbm, ub, su, j).start()

    j = i + LA

    @pl.when(j < NCH - 1)
    def _start_full():
        _u_copy(u_hbm, ub, su, j).start()

    @pl.when(j == NCH - 1)
    def _start_last():
        _u_copy_last(u_hbm, ubl, sul).start()

    def update(key, pb_c, lane):
        local_min = jnp.min(key, axis=1, keepdims=True)
        w = key == local_min
        local_arg = jnp.min(jnp.where(w, lane, _I32MAX),
                            axis=1, keepdims=True)
        local_prob = jnp.max(jnp.where(w, pb_c, -jnp.inf),
                             axis=1, keepdims=True)
        upd = local_min < bk[...]
        bk[...] = jnp.where(upd, local_min, bk[...])
        bi[...] = jnp.where(upd, i * CW + local_arg, bi[...])
        bp[...] = jnp.where(upd, local_prob, bp[...])

    @pl.when(i < NCH - 1)
    def _compute_full():
        _u_copy(u_hbm, ub, su, i).wait()
        pb = pb_ref[...]
        lane = jax.lax.broadcasted_iota(jnp.int32, (1, CW), 1)
        key = -jnp.log(ub[slot]) * jnp.exp(-pb)
        update(key, pb, lane)

    @pl.when(i == NCH - 1)
    def _compute_last():
        _u_copy_last(u_hbm, ubl, sul).wait()
        pb = pb_ref[...][:, :LAST]
        lane = jax.lax.broadcasted_iota(jnp.int32, (1, LAST), 1)
        key = -jnp.log(ubl[...]) * jnp.exp(-pb)
        update(key, pb, lane)

        pf = pfull_ref[...]
        mx = jnp.max(pf, keepdims=True).reshape(1, 1)
        s = jnp.sum(jnp.exp(pf - mx), keepdims=True).reshape(1, 1)
        lse = mx + jnp.log(s)
        y_ref[...] = bi[...]
        lp_ref[...] = bp[...] - lse


def _tc_argmin(prob2, u):
    row_spec = pl.BlockSpec((1, CW), lambda i: (0, i))
    full_spec = pl.BlockSpec((1, D), lambda i: (0, 0))
    hbm_spec = pl.BlockSpec(memory_space=pltpu.MemorySpace.HBM)
    return pl.pallas_call(
        _tc_body,
        grid=(NCH,),
        in_specs=[row_spec, full_spec, hbm_spec],
        out_specs=[
            pl.BlockSpec((B, 1), lambda i: (0, 0)),
            pl.BlockSpec((B, 1), lambda i: (0, 0)),
        ],
        out_shape=[
            jax.ShapeDtypeStruct((B, 1), jnp.int32),
            jax.ShapeDtypeStruct((B, 1), jnp.float32),
        ],
        scratch_shapes=[
            pltpu.VMEM((NBUF, B, CW), jnp.float32),
            pltpu.VMEM((B, LAST), jnp.float32),
            pltpu.SemaphoreType.DMA((NBUF,)),
            pltpu.SemaphoreType.DMA,
            pltpu.VMEM((B, 1), jnp.float32),
            pltpu.VMEM((B, 1), jnp.int32),
            pltpu.VMEM((B, 1), jnp.float32),
        ],
    )(prob2, prob2, u)


RB = 16
CB = 1024
DSC = (D // CB) * CB
SC_LANES = 16

def _sc_z(m2, ls2, eps):
    @pl.kernel(
        out_type=jax.ShapeDtypeStruct((B, D + 1), jnp.float32),
        mesh=plsc.VectorSubcoreMesh(
            core_axis_name="core", subcore_axis_name="subcore"),
        compiler_params=pltpu.CompilerParams(use_tc_tiling_on_sc=True),
    )
    def zkernel(m_hbm, ls_hbm, e_hbm, o_hbm):
        def body(m_v, ls_v, e_v, o_v):
            @plsc.parallel_loop(0, CB, SC_LANES, unroll=8)
            def _(c):
                m16 = m_v.at[0, pl.ds(c, SC_LANES)][...]
                s16 = jnp.exp(ls_v.at[0, pl.ds(c, SC_LANES)][...])
                for r in range(RB):
                    o_v.at[r, pl.ds(c, SC_LANES)][...] = (
                        m16 + s16 * e_v.at[r, pl.ds(c, SC_LANES)][...])

        pltpu.emit_pipeline(
            body,
            grid=(B // RB, DSC // CB),
            in_specs=[
                pl.BlockSpec((1, CB), index_map=lambda i, j: (0, j)),
                pl.BlockSpec((1, CB), index_map=lambda i, j: (0, j)),
                pl.BlockSpec((RB, CB), index_map=lambda i, j: (i, j)),
            ],
            out_specs=[
                pl.BlockSpec((RB, CB), index_map=lambda i, j: (i, j)),
            ],
            core_axis_name=("core", "subcore"),
            dimension_semantics=(pltpu.PARALLEL, pltpu.PARALLEL),
        )(m_hbm, ls_hbm, e_hbm, o_hbm)

    return zkernel(m2, ls2, eps)


_TBLK = 1024
_TCOLBLK = DSC // _TBLK
_TOFF = D - DSC


def _col_body(oz_ref, m_ref, ls_ref, e_ref, lp_ref, out_ref):
    out_ref[...] = m_ref[...] + jnp.exp(ls_ref[...]) * e_ref[...]
    out_ref[:, _TOFF:_TOFF + 1] = lp_ref[...]


def _write_col(out_z, m2, ls2, eps, logpy):
    oblk = pl.BlockSpec((B, _TBLK), lambda i: (0, _TCOLBLK))
    rblk = pl.BlockSpec((1, _TBLK), lambda i: (0, _TCOLBLK))
    return pl.pallas_call(
        _col_body,
        grid=(1,),
        in_specs=[oblk, rblk, rblk,
                  pl.BlockSpec((B, _TBLK), lambda i: (0, _TCOLBLK)),
                  pl.BlockSpec((B, 1), lambda i: (0, 0))],
        out_specs=oblk,
        out_shape=jax.ShapeDtypeStruct((B, D + 1), jnp.float32),
        input_output_aliases={0: 0},
    )(out_z, m2, ls2, eps, logpy)


@jax.jit
def kernel(prob, m_z, log_s_z, u, eps):
    prob2 = prob.reshape(1, D)
    m2 = m_z.reshape(1, D)
    ls2 = log_s_z.reshape(1, D)

    out_z = _sc_z(m2, ls2, eps)
    y2, logpy = _tc_argmin(prob2, u)
    out = _write_col(out_z, m2, ls2, eps, logpy)
    return (y2.reshape(B), out)


# speedup vs baseline: 1.1105x; 1.1105x over previous
import jax
import jax.numpy as jnp
from jax.experimental import pallas as pl
from jax.experimental.pallas import tpu as pltpu

D = 100000
B = 128
CW = 8192
NCH = (D + CW - 1) // CW
LAST = D - (NCH - 1) * CW
NBUF = 3
LA = NBUF - 1
_I32MAX = jnp.iinfo(jnp.int32).max


def _in_copies(u_hbm, e_hbm, ub, eb, su, se, j):
    slot = jax.lax.rem(j, NBUF)
    cu = pltpu.make_async_copy(
        u_hbm.at[:, pl.ds(j * CW, CW)], ub.at[slot], su.at[slot])
    ce = pltpu.make_async_copy(
        e_hbm.at[:, pl.ds(j * CW, CW)], eb.at[slot], se.at[slot])
    return cu, ce


def _in_copies_last(u_hbm, e_hbm, ubl, ebl, sul, sel):
    base = (NCH - 1) * CW
    cu = pltpu.make_async_copy(u_hbm.at[:, pl.ds(base, LAST)], ubl, sul)
    ce = pltpu.make_async_copy(e_hbm.at[:, pl.ds(base, LAST)], ebl, sel)
    return cu, ce


def _out_copy(out_hbm, ob, so, j):
    slot = jax.lax.rem(j, NBUF)
    return pltpu.make_async_copy(
        ob.at[slot], out_hbm.at[:, pl.ds(j * CW, CW)], so.at[slot])


def _out_copy_last(out_hbm, obl, sol):
    base = (NCH - 1) * CW
    return pltpu.make_async_copy(
        obl, out_hbm.at[:, pl.ds(base, LAST + 1)], sol)


def _body(pb_ref, m_ref, ls_ref, pfull_ref, u_hbm, e_hbm,
          out_hbm, y_ref,
          ub, eb, ob, ubl, ebl, obl,
          su, se, so, sul, sel, sol,
          bk, bi, bp):
    i = pl.program_id(0)
    slot = jax.lax.rem(i, NBUF)

    @pl.when(i == 0)
    def _prologue():
        bk[...] = jnp.full((B, 1), jnp.inf, jnp.float32)
        bi[...] = jnp.zeros((B, 1), jnp.int32)
        bp[...] = jnp.zeros((B, 1), jnp.float32)
        for j in range(LA):
            cu, ce = _in_copies(u_hbm, e_hbm, ub, eb, su, se, j)
            cu.start()
            ce.start()

    j = i + LA

    @pl.when(j < NCH - 1)
    def _start_full():
        cu, ce = _in_copies(u_hbm, e_hbm, ub, eb, su, se, j)
        cu.start()
        ce.start()

    @pl.when(j == NCH - 1)
    def _start_last():
        cu, ce = _in_copies_last(u_hbm, e_hbm, ubl, ebl, sul, sel)
        cu.start()
        ce.start()

    @pl.when(jnp.logical_and(i >= NBUF, i < NCH - 1))
    def _drain_out():
        _out_copy(out_hbm, ob, so, i - NBUF).wait()

    def update(key, pb_c, lane):
        local_min = jnp.min(key, axis=1, keepdims=True)
        w = key == local_min
        local_arg = jnp.min(jnp.where(w, lane, _I32MAX),
                            axis=1, keepdims=True)
        local_prob = jnp.max(jnp.where(w, pb_c, -jnp.inf),
                             axis=1, keepdims=True)
        upd = local_min < bk[...]
        bk[...] = jnp.where(upd, local_min, bk[...])
        bi[...] = jnp.where(upd, i * CW + local_arg, bi[...])
        bp[...] = jnp.where(upd, local_prob, bp[...])

    @pl.when(i < NCH - 1)
    def _compute_full():
        cu, ce = _in_copies(u_hbm, e_hbm, ub, eb, su, se, i)
        cu.wait()
        ce.wait()
        pb = pb_ref[...]
        lane = jax.lax.broadcasted_iota(jnp.int32, (1, CW), 1)
        e = -jnp.log(ub[slot])
        key = e * jnp.exp(-pb)
        ob[slot] = m_ref[...] + jnp.exp(ls_ref[...]) * eb[slot]
        update(key, pb, lane)
        _out_copy(out_hbm, ob, so, i).start()

    @pl.when(i == NCH - 1)
    def _compute_last():
        cu, ce = _in_copies_last(u_hbm, e_hbm, ubl, ebl, sul, sel)
        cu.wait()
        ce.wait()
        pb = pb_ref[...][:, :LAST]
        lane = jax.lax.broadcasted_iota(jnp.int32, (1, LAST), 1)
        e = -jnp.log(ubl[...])
        key = e * jnp.exp(-pb)
        obl[:, :LAST] = (m_ref[...][:, :LAST]
                         + jnp.exp(ls_ref[...][:, :LAST]) * ebl[...])
        update(key, pb, lane)

        pf = pfull_ref[...]
        mx = jnp.max(pf, keepdims=True).reshape(1, 1)
        s = jnp.sum(jnp.exp(pf - mx), keepdims=True).reshape(1, 1)
        lse = mx + jnp.log(s)
        y_ref[...] = bi[...]
        obl[:, LAST:LAST + 1] = bp[...] - lse
        _out_copy_last(out_hbm, obl, sol).start()

        for k in range(NCH - 1 - NBUF, NCH - 1):
            _out_copy(out_hbm, ob, so, k).wait()
        _out_copy_last(out_hbm, obl, sol).wait()


@jax.jit
def kernel(prob, m_z, log_s_z, u, eps):
    prob2 = prob.reshape(1, D)
    m2 = m_z.reshape(1, D)
    ls2 = log_s_z.reshape(1, D)

    row_spec = pl.BlockSpec((1, CW), lambda i: (0, i))
    full_spec = pl.BlockSpec((1, D), lambda i: (0, 0))
    any_spec = pl.BlockSpec(memory_space=pltpu.MemorySpace.HBM)

    out, y2 = pl.pallas_call(
        _body,
        grid=(NCH,),
        in_specs=[row_spec, row_spec, row_spec, full_spec,
                  any_spec, any_spec],
        out_specs=[
            any_spec,
            pl.BlockSpec((B, 1), lambda i: (0, 0)),
        ],
        out_shape=[
            jax.ShapeDtypeStruct((B, D + 1), jnp.float32),
            jax.ShapeDtypeStruct((B, 1), jnp.int32),
        ],
        scratch_shapes=[
            pltpu.VMEM((NBUF, B, CW), jnp.float32),
            pltpu.VMEM((NBUF, B, CW), jnp.float32),
            pltpu.VMEM((NBUF, B, CW), jnp.float32),
            pltpu.VMEM((B, LAST), jnp.float32),
            pltpu.VMEM((B, LAST), jnp.float32),
            pltpu.VMEM((B, LAST + 1), jnp.float32),
            pltpu.SemaphoreType.DMA((NBUF,)),
            pltpu.SemaphoreType.DMA((NBUF,)),
            pltpu.SemaphoreType.DMA((NBUF,)),
            pltpu.SemaphoreType.DMA,
            pltpu.SemaphoreType.DMA,
            pltpu.SemaphoreType.DMA,
            pltpu.VMEM((B, 1), jnp.float32),
            pltpu.VMEM((B, 1), jnp.int32),
            pltpu.VMEM((B, 1), jnp.float32),
        ],
    )(prob2, m2, ls2, prob2, u, eps)
    return (y2.reshape(B), out)
